# Initial kernel scaffold; baseline (speedup 1.0000x reference)
import jax
import jax.numpy as jnp
from jax.experimental import pallas as pl


def kernel(output, target, one_hot):
    raise NotImplementedError("write your pallas kernel here")


# speedup vs baseline: 8.8789x; 8.8789x over previous
import math
import functools

import jax
import jax.numpy as jnp
from jax import lax
from jax.experimental import pallas as pl
from jax.experimental.pallas import tpu as pltpu

SIZE_ = 32000
PAD_ = 0
SMOOTH_ = 0.1
CONF_ = 1.0 - SMOOTH_
SVAL_ = SMOOTH_ / (SIZE_ - 2)
K_ = (SIZE_ - 2) * SVAL_ * math.log(SVAL_) + CONF_ * math.log(CONF_)

B_ = 2048
BC_ = 1280


def _loss_body(out_ref, t_ref, acc_ref):
    j = pl.program_id(0)
    out_blk = out_ref[...]
    t_blk = t_ref[...]
    nonpad = t_blk != PAD_

    col0 = j * BC_
    colids = col0 + lax.broadcasted_iota(jnp.int32, (B_, BC_), 1)
    w = jnp.where(colids == t_blk, CONF_, SVAL_)
    w = jnp.where(colids == 0, 0.0, w)
    w = jnp.where(nonpad, w, 0.0)
    partial = jnp.sum(out_blk * w)

    @pl.when(j == 0)
    def _init():
        cnt = jnp.sum(nonpad.astype(jnp.float32))
        acc_ref[0, 0] = K_ * cnt

    acc_ref[0, 0] = acc_ref[0, 0] - partial


@jax.jit
def kernel(output, target, one_hot):
    del one_hot
    t2 = target.astype(jnp.int32).reshape(B_, 1)
    acc = pl.pallas_call(
        _loss_body,
        grid=(SIZE_ // BC_,),
        in_specs=[
            pl.BlockSpec((B_, BC_), lambda j: (0, j)),
            pl.BlockSpec((B_, 1), lambda j: (0, 0)),
        ],
        out_specs=pl.BlockSpec(
            (1, 1), lambda j: (0, 0), memory_space=pltpu.SMEM
        ),
        out_shape=jax.ShapeDtypeStruct((1, 1), jnp.float32),
    )(output, t2)
    return acc[0, 0]
